# baseline (speedup 1.0000x reference)
import functools

import jax
import jax.numpy as jnp
from jax.experimental import pallas as pl
from jax.experimental.pallas import tpu as pltpu


def _top2_combine(logits, e):
    iota = jax.lax.broadcasted_iota(jnp.int32, logits.shape, 1)
    m1 = jnp.max(logits, axis=-1, keepdims=True)
    i1 = jnp.min(jnp.where(logits == m1, iota, e), axis=-1, keepdims=True)
    masked = jnp.where(iota == i1, -jnp.inf, logits)
    m2 = jnp.max(masked, axis=-1, keepdims=True)
    i2 = jnp.min(jnp.where(masked == m2, iota, e), axis=-1, keepdims=True)
    p1 = jax.nn.sigmoid(m1 - m2)
    p2 = 1.0 - p1
    return jnp.where(iota == i1, p1, 0.0) + jnp.where(iota == i2, p2, 0.0)


def _moe_kernel(x_ref, rw_ref, brw_ref, wbank_ref, bbank_ref, out_ref,
                *, n_experts):
    x = x_ref[...]
    w_logits = jnp.dot(x, rw_ref[...], preferred_element_type=jnp.float32)
    b_logits = jnp.dot(x, brw_ref[...], preferred_element_type=jnp.float32)
    w_comb = _top2_combine(w_logits, n_experts)
    b_comb = _top2_combine(b_logits, n_experts)

    acc = jnp.dot(b_comb, bbank_ref[...], preferred_element_type=jnp.float32)
    for e in range(n_experts):
        y = jnp.dot(x, wbank_ref[e], preferred_element_type=jnp.float32)
        acc = acc + w_comb[:, e][:, None] * y
    out_ref[...] = acc


@jax.jit
def kernel(input_batch, router_w, bias_router_w, weight_bank, bias_bank):
    b, d = input_batch.shape
    e, _, o = weight_bank.shape
    oc = o // 2
    return pl.pallas_call(
        functools.partial(_moe_kernel, n_experts=e),
        grid=(2,),
        out_shape=jax.ShapeDtypeStruct((b, o), jnp.float32),
        in_specs=[
            pl.BlockSpec((b, d), lambda c: (0, 0)),
            pl.BlockSpec((d, e), lambda c: (0, 0)),
            pl.BlockSpec((d, e), lambda c: (0, 0)),
            pl.BlockSpec((e, d, oc), lambda c: (0, 0, c)),
            pl.BlockSpec((e, oc), lambda c: (0, c)),
        ],
        out_specs=pl.BlockSpec((b, oc), lambda c: (0, c)),
        compiler_params=pltpu.CompilerParams(
            dimension_semantics=("parallel",),
        ),
    )(input_batch, router_w, bias_router_w, weight_bank, bias_bank)


# speedup vs baseline: 1.3531x; 1.3531x over previous
import functools

import jax
import jax.numpy as jnp
from jax.experimental import pallas as pl
from jax.experimental.pallas import tpu as pltpu

_CHUNK = 4


def _top2_combine(logits, e):
    iota = jax.lax.broadcasted_iota(jnp.int32, logits.shape, 1)
    m1 = jnp.max(logits, axis=-1, keepdims=True)
    i1 = jnp.min(jnp.where(logits == m1, iota, e), axis=-1, keepdims=True)
    masked = jnp.where(iota == i1, -jnp.inf, logits)
    m2 = jnp.max(masked, axis=-1, keepdims=True)
    i2 = jnp.min(jnp.where(masked == m2, iota, e), axis=-1, keepdims=True)
    p1 = jax.nn.sigmoid(m1 - m2)
    p2 = 1.0 - p1
    return jnp.where(iota == i1, p1, 0.0) + jnp.where(iota == i2, p2, 0.0)


def _moe_kernel(x_ref, rw_ref, brw_ref, wbank_hbm, bbank_ref, out_ref,
                wbuf, sems, *, n_experts):
    n_chunks = n_experts // _CHUNK
    for c in range(n_chunks):
        sl = pl.ds(c * _CHUNK, _CHUNK)
        pltpu.make_async_copy(wbank_hbm.at[sl], wbuf.at[sl], sems.at[c]).start()

    x = x_ref[...]
    w_logits = jnp.dot(x, rw_ref[...], preferred_element_type=jnp.float32)
    b_logits = jnp.dot(x, brw_ref[...], preferred_element_type=jnp.float32)
    w_comb = _top2_combine(w_logits, n_experts)
    b_comb = _top2_combine(b_logits, n_experts)

    acc = jnp.dot(b_comb, bbank_ref[...], preferred_element_type=jnp.float32)
    for c in range(n_chunks):
        sl = pl.ds(c * _CHUNK, _CHUNK)
        pltpu.make_async_copy(wbank_hbm.at[sl], wbuf.at[sl], sems.at[c]).wait()
        for e in range(c * _CHUNK, (c + 1) * _CHUNK):
            y = jnp.dot(x, wbuf[e], preferred_element_type=jnp.float32)
            acc = acc + w_comb[:, e][:, None] * y
    out_ref[...] = acc


@jax.jit
def kernel(input_batch, router_w, bias_router_w, weight_bank, bias_bank):
    b, d = input_batch.shape
    e, _, o = weight_bank.shape
    return pl.pallas_call(
        functools.partial(_moe_kernel, n_experts=e),
        out_shape=jax.ShapeDtypeStruct((b, o), jnp.float32),
        in_specs=[
            pl.BlockSpec((b, d), lambda: (0, 0)),
            pl.BlockSpec((d, e), lambda: (0, 0)),
            pl.BlockSpec((d, e), lambda: (0, 0)),
            pl.BlockSpec(memory_space=pl.ANY),
            pl.BlockSpec((e, o), lambda: (0, 0)),
        ],
        out_specs=pl.BlockSpec((b, o), lambda: (0, 0)),
        scratch_shapes=[
            pltpu.VMEM((e, d, o), jnp.float32),
            pltpu.SemaphoreType.DMA((e // _CHUNK,)),
        ],
    )(input_batch, router_w, bias_router_w, weight_bank, bias_bank)
